# baseline (speedup 1.0000x reference)
import functools

import jax
import jax.numpy as jnp
from jax.experimental import pallas as pl
from jax.experimental.pallas import tpu as pltpu

_D = 128
_BE = 3200
_EPS = 1e-5


def _mm_bias_body(x_ref, w_ref, b_ref, o_ref):
    o_ref[...] = (
        jnp.dot(x_ref[...], w_ref[...], preferred_element_type=jnp.float32)
        + b_ref[...]
    )


def _mm_bias(x, w, b, be):
    m, k = x.shape
    dp = w.shape[1]
    return pl.pallas_call(
        _mm_bias_body,
        grid=(m // be,),
        in_specs=[
            pl.BlockSpec((be, k), lambda i: (i, 0)),
            pl.BlockSpec((k, dp), lambda i: (0, 0)),
            pl.BlockSpec((1, dp), lambda i: (0, 0)),
        ],
        out_specs=pl.BlockSpec((be, dp), lambda i: (i, 0)),
        out_shape=jax.ShapeDtypeStruct((m, dp), jnp.float32),
    )(x, w, b)


def _h_update_body(h_ref, hu_ref, agg_ref, o_ref, mean_ref):
    agg = agg_ref[...]
    agg = jnp.where(jnp.isfinite(agg), agg, 0.0)
    x = hu_ref[...] + agg
    m = jnp.mean(x, axis=0, keepdims=True)
    v = jnp.mean((x - m) ** 2, axis=0, keepdims=True)
    out = h_ref[...] + jax.nn.relu((x - m) / jnp.sqrt(v + _EPS))
    o_ref[...] = out
    mean_ref[...] = jnp.mean(out, axis=0, keepdims=True)


def _h_update(h, hu, agg):
    n = h.shape[0]
    return pl.pallas_call(
        _h_update_body,
        in_specs=[
            pl.BlockSpec((n, _D), lambda: (0, 0)),
            pl.BlockSpec((n, _D), lambda: (0, 0)),
            pl.BlockSpec((n, _D), lambda: (0, 0)),
        ],
        out_specs=[
            pl.BlockSpec((n, _D), lambda: (0, 0)),
            pl.BlockSpec((1, _D), lambda: (0, 0)),
        ],
        out_shape=[
            jax.ShapeDtypeStruct((n, _D), jnp.float32),
            jax.ShapeDtypeStruct((1, _D), jnp.float32),
        ],
    )(h, hu, agg)


def _e1_body(e_ref, s_ref, a_ref, t_ref, stats_ref, acc_ref):
    i = pl.program_id(0)
    t = (
        jnp.dot(e_ref[...], a_ref[...], preferred_element_type=jnp.float32)
        + s_ref[...]
    )
    t_ref[...] = t

    @pl.when(i == 0)
    def _():
        acc_ref[...] = jnp.zeros_like(acc_ref)

    acc_ref[0:1, :] += jnp.sum(t, axis=0, keepdims=True)
    acc_ref[1:2, :] += jnp.sum(t * t, axis=0, keepdims=True)

    @pl.when(i == pl.num_programs(0) - 1)
    def _():
        stats_ref[...] = acc_ref[0:2, :]


def _e1(e, s, a):
    ne = e.shape[0]
    return pl.pallas_call(
        _e1_body,
        grid=(ne // _BE,),
        in_specs=[
            pl.BlockSpec((_BE, _D), lambda i: (i, 0)),
            pl.BlockSpec((_BE, _D), lambda i: (i, 0)),
            pl.BlockSpec((_D, _D), lambda i: (0, 0)),
        ],
        out_specs=[
            pl.BlockSpec((_BE, _D), lambda i: (i, 0)),
            pl.BlockSpec((2, _D), lambda i: (0, 0)),
        ],
        out_shape=[
            jax.ShapeDtypeStruct((ne, _D), jnp.float32),
            jax.ShapeDtypeStruct((2, _D), jnp.float32),
        ],
        scratch_shapes=[pltpu.VMEM((8, _D), jnp.float32)],
    )(e, s, a)


def _e2_body(ne, e_ref, t_ref, stats_ref, o_ref):
    mean = stats_ref[0:1, :] / ne
    var = stats_ref[1:2, :] / ne - mean * mean
    o_ref[...] = e_ref[...] + jax.nn.relu(
        (t_ref[...] - mean) * jax.lax.rsqrt(var + _EPS)
    )


def _e2(e, t, stats):
    ne = e.shape[0]
    return pl.pallas_call(
        functools.partial(_e2_body, float(ne)),
        grid=(ne // _BE,),
        in_specs=[
            pl.BlockSpec((_BE, _D), lambda i: (i, 0)),
            pl.BlockSpec((_BE, _D), lambda i: (i, 0)),
            pl.BlockSpec((2, _D), lambda i: (0, 0)),
        ],
        out_specs=pl.BlockSpec((_BE, _D), lambda i: (i, 0)),
        out_shape=jax.ShapeDtypeStruct((ne, _D), jnp.float32),
    )(e, t, stats)


def _final_body(g_ref, w1_ref, b1_ref, w2_ref, b2_ref, wf_ref, bf_ref, o_ref):
    x = jax.nn.relu(g_ref[...])
    y = jax.nn.relu(
        jnp.dot(x, w1_ref[...], preferred_element_type=jnp.float32) + b1_ref[...]
    )
    z = jax.nn.relu(
        jnp.dot(y, w2_ref[...], preferred_element_type=jnp.float32) + b2_ref[...]
    )
    o = jnp.sum(z * wf_ref[...], axis=1) + bf_ref[0, 0]
    o_ref[...] = jax.nn.sigmoid(o).reshape(o_ref.shape)


_BF = 4096


def _final(g, w1, b1, w2, b2, wf, bf):
    ne = g.shape[0]
    return pl.pallas_call(
        _final_body,
        grid=(pl.cdiv(ne, _BF),),
        in_specs=[
            pl.BlockSpec((_BF, _D), lambda i: (i, 0)),
            pl.BlockSpec((_D, _D), lambda i: (0, 0)),
            pl.BlockSpec((1, _D), lambda i: (0, 0)),
            pl.BlockSpec((_D, _D), lambda i: (0, 0)),
            pl.BlockSpec((1, _D), lambda i: (0, 0)),
            pl.BlockSpec((1, _D), lambda i: (0, 0)),
            pl.BlockSpec((1, 1), lambda i: (0, 0)),
        ],
        out_specs=pl.BlockSpec((_BF // _D, _D), lambda i: (i, 0)),
        out_shape=jax.ShapeDtypeStruct((ne // _D, _D), jnp.float32),
    )(g, w1, b1, w2, b2, wf, bf).reshape(ne)


def kernel(h, e, edge_index, emb_n_w, emb_n_b, emb_e_w, emb_e_b, U_w, V_w,
           A_w, B_w, C_w, W0_w, W0_b, Wk_w, Wk_b, Wf_w, Wf_b):
    n = h.shape[0]
    src = edge_index[0]
    dst = edge_index[1]

    h = _mm_bias(h, emb_n_w.T, emb_n_b[None], n)
    e = _mm_bias(e, emb_e_w.T, emb_e_b[None], _BE)

    hmean = None
    for l in range(3):
        last = l == 2
        if last:
            wcat = jnp.concatenate([V_w[l].T, U_w[l].T], axis=1)
        else:
            wcat = jnp.concatenate(
                [V_w[l].T, U_w[l].T, B_w[l].T, C_w[l].T], axis=1
            )
        hcat = _mm_bias(h, wcat, jnp.zeros((1, wcat.shape[1]), jnp.float32), n)
        hv = hcat[:, :_D]
        hu = hcat[:, _D : 2 * _D]
        m = hv[src] * jax.nn.sigmoid(e)
        agg = jax.ops.segment_max(m, dst, num_segments=n)
        h_new, hmean = _h_update(h, hu, agg)
        if not last:
            hb = hcat[:, 2 * _D : 3 * _D]
            hc = hcat[:, 3 * _D :]
            s = hb[dst] + hc[src]
            t, stats = _e1(e, s, A_w[l].T)
            e = _e2(e, t, stats)
        h = h_new

    w0m = W0_w[:, :_D]
    w0i = W0_w[:, _D : 2 * _D]
    w0j = W0_w[:, 2 * _D :]
    c = hmean @ w0m.T + W0_b[None]
    pq = _mm_bias(
        h,
        jnp.concatenate([w0i.T, w0j.T], axis=1),
        jnp.concatenate([c, jnp.zeros((1, _D), jnp.float32)], axis=1),
        n,
    )
    p = pq[:, :_D]
    q = pq[:, _D:]
    g = p[src] + q[dst]
    return _final(
        g, Wk_w[0].T, Wk_b[0][None], Wk_w[1].T, Wk_b[1][None],
        Wf_w, Wf_b[None],
    )


# speedup vs baseline: 1.3275x; 1.3275x over previous
import functools

import jax
import jax.numpy as jnp
from jax import lax
from jax.experimental import pallas as pl
from jax.experimental.pallas import tpu as pltpu
from jax.experimental.pallas import tpu_sc as plsc

_D = 128
_BE = 3200
_EPS = 1e-5

_SC_INFO = plsc.get_sparse_core_info()
_NC = _SC_INFO.num_cores
_NS = _SC_INFO.num_subcores
_NW = _NC * _NS


def _sc_mesh():
    return plsc.VectorSubcoreMesh(core_axis_name="c", subcore_axis_name="s")


def _gather_sum(ta, tb, ia, ib):
    e = ia.shape[0]
    per_w = e // _NW
    chunk = 400
    nch = per_w // chunk

    @functools.partial(
        pl.kernel,
        mesh=_sc_mesh(),
        out_type=jax.ShapeDtypeStruct((e, _D), jnp.float32),
        scratch_types=[
            pltpu.VMEM((chunk,), jnp.int32),
            pltpu.VMEM((chunk,), jnp.int32),
            pltpu.VMEM((chunk, _D), jnp.float32),
            pltpu.VMEM((chunk, _D), jnp.float32),
            pltpu.SemaphoreType.DMA,
            pltpu.SemaphoreType.DMA,
        ],
    )
    def k(ta_hbm, tb_hbm, ia_hbm, ib_hbm, out_hbm, ia_v, ib_v, ra_v, rb_v,
          sem_a, sem_b):
        wid = lax.axis_index("s") * _NC + lax.axis_index("c")
        base = wid * per_w

        def body(c, carry):
            off = base + c * chunk
            pltpu.sync_copy(ia_hbm.at[pl.ds(off, chunk)], ia_v)
            pltpu.sync_copy(ib_hbm.at[pl.ds(off, chunk)], ib_v)
            cpa = pltpu.async_copy(ta_hbm.at[ia_v], ra_v, sem_a)
            cpb = pltpu.async_copy(tb_hbm.at[ib_v], rb_v, sem_b)
            cpa.wait()
            cpb.wait()

            def add_row(r, carry2):
                for kk in range(_D // 16):
                    sl = pl.ds(kk * 16, 16)
                    plsc.addupdate(ra_v.at[r, sl], rb_v[r, sl])
                return carry2

            lax.fori_loop(0, chunk, add_row, 0)
            pltpu.sync_copy(ra_v, out_hbm.at[pl.ds(off, chunk)])
            return carry

        lax.fori_loop(0, nch, body, 0)

    return k(ta, tb, ia, ib)


def _mm_bias_body(x_ref, w_ref, b_ref, o_ref):
    o_ref[...] = (
        jnp.dot(x_ref[...], w_ref[...], preferred_element_type=jnp.float32)
        + b_ref[...]
    )


def _mm_bias(x, w, b, be):
    m, k = x.shape
    dp = w.shape[1]
    return pl.pallas_call(
        _mm_bias_body,
        grid=(m // be,),
        in_specs=[
            pl.BlockSpec((be, k), lambda i: (i, 0)),
            pl.BlockSpec((k, dp), lambda i: (0, 0)),
            pl.BlockSpec((1, dp), lambda i: (0, 0)),
        ],
        out_specs=pl.BlockSpec((be, dp), lambda i: (i, 0)),
        out_shape=jax.ShapeDtypeStruct((m, dp), jnp.float32),
    )(x, w, b)


def _h_update_body(h_ref, hu_ref, agg_ref, o_ref, mean_ref):
    agg = agg_ref[...]
    agg = jnp.where(jnp.isfinite(agg), agg, 0.0)
    x = hu_ref[...] + agg
    m = jnp.mean(x, axis=0, keepdims=True)
    v = jnp.mean((x - m) ** 2, axis=0, keepdims=True)
    out = h_ref[...] + jax.nn.relu((x - m) / jnp.sqrt(v + _EPS))
    o_ref[...] = out
    mean_ref[...] = jnp.mean(out, axis=0, keepdims=True)


def _h_update(h, hu, agg):
    n = h.shape[0]
    return pl.pallas_call(
        _h_update_body,
        in_specs=[
            pl.BlockSpec((n, _D), lambda: (0, 0)),
            pl.BlockSpec((n, _D), lambda: (0, 0)),
            pl.BlockSpec((n, _D), lambda: (0, 0)),
        ],
        out_specs=[
            pl.BlockSpec((n, _D), lambda: (0, 0)),
            pl.BlockSpec((1, _D), lambda: (0, 0)),
        ],
        out_shape=[
            jax.ShapeDtypeStruct((n, _D), jnp.float32),
            jax.ShapeDtypeStruct((1, _D), jnp.float32),
        ],
    )(h, hu, agg)


def _e1_body(e_ref, s_ref, a_ref, t_ref, stats_ref, acc_ref):
    i = pl.program_id(0)
    t = (
        jnp.dot(e_ref[...], a_ref[...], preferred_element_type=jnp.float32)
        + s_ref[...]
    )
    t_ref[...] = t

    @pl.when(i == 0)
    def _():
        acc_ref[...] = jnp.zeros_like(acc_ref)

    acc_ref[0:1, :] += jnp.sum(t, axis=0, keepdims=True)
    acc_ref[1:2, :] += jnp.sum(t * t, axis=0, keepdims=True)

    @pl.when(i == pl.num_programs(0) - 1)
    def _():
        stats_ref[...] = acc_ref[0:2, :]


def _e1(e, s, a):
    ne = e.shape[0]
    return pl.pallas_call(
        _e1_body,
        grid=(ne // _BE,),
        in_specs=[
            pl.BlockSpec((_BE, _D), lambda i: (i, 0)),
            pl.BlockSpec((_BE, _D), lambda i: (i, 0)),
            pl.BlockSpec((_D, _D), lambda i: (0, 0)),
        ],
        out_specs=[
            pl.BlockSpec((_BE, _D), lambda i: (i, 0)),
            pl.BlockSpec((2, _D), lambda i: (0, 0)),
        ],
        out_shape=[
            jax.ShapeDtypeStruct((ne, _D), jnp.float32),
            jax.ShapeDtypeStruct((2, _D), jnp.float32),
        ],
        scratch_shapes=[pltpu.VMEM((8, _D), jnp.float32)],
    )(e, s, a)


def _e2_body(ne, e_ref, t_ref, stats_ref, o_ref):
    mean = stats_ref[0:1, :] / ne
    var = stats_ref[1:2, :] / ne - mean * mean
    o_ref[...] = e_ref[...] + jax.nn.relu(
        (t_ref[...] - mean) * jax.lax.rsqrt(var + _EPS)
    )


def _e2(e, t, stats):
    ne = e.shape[0]
    return pl.pallas_call(
        functools.partial(_e2_body, float(ne)),
        grid=(ne // _BE,),
        in_specs=[
            pl.BlockSpec((_BE, _D), lambda i: (i, 0)),
            pl.BlockSpec((_BE, _D), lambda i: (i, 0)),
            pl.BlockSpec((2, _D), lambda i: (0, 0)),
        ],
        out_specs=pl.BlockSpec((_BE, _D), lambda i: (i, 0)),
        out_shape=jax.ShapeDtypeStruct((ne, _D), jnp.float32),
    )(e, t, stats)


def _final_body(g_ref, w1_ref, b1_ref, w2_ref, b2_ref, wf_ref, bf_ref, o_ref):
    x = jax.nn.relu(g_ref[...])
    y = jax.nn.relu(
        jnp.dot(x, w1_ref[...], preferred_element_type=jnp.float32) + b1_ref[...]
    )
    z = jax.nn.relu(
        jnp.dot(y, w2_ref[...], preferred_element_type=jnp.float32) + b2_ref[...]
    )
    o = jnp.sum(z * wf_ref[...], axis=1) + bf_ref[0, 0]
    o_ref[...] = jax.nn.sigmoid(o).reshape(o_ref.shape)


_BF = 4096


def _final(g, w1, b1, w2, b2, wf, bf):
    ne = g.shape[0]
    return pl.pallas_call(
        _final_body,
        grid=(pl.cdiv(ne, _BF),),
        in_specs=[
            pl.BlockSpec((_BF, _D), lambda i: (i, 0)),
            pl.BlockSpec((_D, _D), lambda i: (0, 0)),
            pl.BlockSpec((1, _D), lambda i: (0, 0)),
            pl.BlockSpec((_D, _D), lambda i: (0, 0)),
            pl.BlockSpec((1, _D), lambda i: (0, 0)),
            pl.BlockSpec((1, _D), lambda i: (0, 0)),
            pl.BlockSpec((1, 1), lambda i: (0, 0)),
        ],
        out_specs=pl.BlockSpec((_BF // _D, _D), lambda i: (i, 0)),
        out_shape=jax.ShapeDtypeStruct((ne // _D, _D), jnp.float32),
    )(g, w1, b1, w2, b2, wf, bf).reshape(ne)


def kernel(h, e, edge_index, emb_n_w, emb_n_b, emb_e_w, emb_e_b, U_w, V_w,
           A_w, B_w, C_w, W0_w, W0_b, Wk_w, Wk_b, Wf_w, Wf_b):
    n = h.shape[0]
    src = edge_index[0]
    dst = edge_index[1]

    h = _mm_bias(h, emb_n_w.T, emb_n_b[None], n)
    e = _mm_bias(e, emb_e_w.T, emb_e_b[None], _BE)

    hmean = None
    for l in range(3):
        last = l == 2
        if last:
            wcat = jnp.concatenate([V_w[l].T, U_w[l].T], axis=1)
        else:
            wcat = jnp.concatenate(
                [V_w[l].T, U_w[l].T, B_w[l].T, C_w[l].T], axis=1
            )
        hcat = _mm_bias(h, wcat, jnp.zeros((1, wcat.shape[1]), jnp.float32), n)
        hv = hcat[:, :_D]
        hu = hcat[:, _D : 2 * _D]
        m = hv[src] * jax.nn.sigmoid(e)
        agg = jax.ops.segment_max(m, dst, num_segments=n)
        h_new, hmean = _h_update(h, hu, agg)
        if not last:
            hb = hcat[:, 2 * _D : 3 * _D]
            hc = hcat[:, 3 * _D :]
            s = _gather_sum(hb, hc, dst, src)
            t, stats = _e1(e, s, A_w[l].T)
            e = _e2(e, t, stats)
        h = h_new

    w0m = W0_w[:, :_D]
    w0i = W0_w[:, _D : 2 * _D]
    w0j = W0_w[:, 2 * _D :]
    c = hmean @ w0m.T + W0_b[None]
    pq = _mm_bias(
        h,
        jnp.concatenate([w0i.T, w0j.T], axis=1),
        jnp.concatenate([c, jnp.zeros((1, _D), jnp.float32)], axis=1),
        n,
    )
    p = pq[:, :_D]
    q = pq[:, _D:]
    g = _gather_sum(p, q, src, dst)
    return _final(
        g, Wk_w[0].T, Wk_b[0][None], Wk_w[1].T, Wk_b[1][None],
        Wf_w, Wf_b[None],
    )
